# baseline (speedup 1.0000x reference)
import functools

import jax
import jax.numpy as jnp
from jax import lax
from jax.experimental import pallas as pl
from jax.experimental.pallas import tpu as pltpu
from jax.experimental.pallas import tpu_sc as plsc

_NC = 2
_NS = 16
_NW = _NC * _NS
_LANES = 16


def kernel(W_in, indices):
    V, D = W_in.shape
    B, L = indices.shape
    C = 128
    BPT = B // _NW
    NBB = BPT // C
    nch = L * NBB
    NBUF = 4
    assert BPT * _NW == B and NBB * C == BPT

    idxT = jnp.swapaxes(indices, 0, 1)

    mesh = plsc.VectorSubcoreMesh(core_axis_name="c", subcore_axis_name="s")

    @functools.partial(
        pl.kernel,
        mesh=mesh,
        out_type=jax.ShapeDtypeStruct((L, D, B), jnp.float32),
        scratch_types=[
            pltpu.VMEM((L, BPT), jnp.int32),
            pltpu.VMEM((NBUF, C, D), jnp.float32),
            pltpu.VMEM((NBUF, D, C), jnp.float32),
            [pltpu.SemaphoreType.DMA] * NBUF,
            [pltpu.SemaphoreType.DMA] * NBUF,
        ],
        compiler_params=pltpu.CompilerParams(
            use_tc_tiling_on_sc=False, needs_layout_passes=False
        ),
    )
    def gather_kernel(table, idx, out, idx_v, rows_v, tr_v, gsem, ssem):
        wid = lax.axis_index("s") * _NC + lax.axis_index("c")
        col0 = wid * BPT
        pltpu.sync_copy(idx.at[:, pl.ds(col0, BPT)], idx_v)

        row_vecs = [
            lax.iota(jnp.int32, _LANES) + jb * _LANES
            for jb in range(C // _LANES)
        ]

        def issue_gather(q, b):
            l = q // NBB
            bb = q % NBB
            pltpu.async_copy(
                table.at[idx_v.at[l, pl.ds(bb * C, C)]], rows_v.at[b], gsem[b]
            )

        def wait_gather(b):
            pltpu.make_async_copy(
                table.at[idx_v.at[0, pl.ds(0, C)]], rows_v.at[b], gsem[b]
            ).wait()

        def issue_store(q, b):
            l = q // NBB
            bb = q % NBB
            pltpu.async_copy(
                tr_v.at[b], out.at[l, :, pl.ds(col0 + bb * C, C)], ssem[b]
            )

        def wait_store(b):
            pltpu.make_async_copy(
                tr_v.at[b], out.at[0, :, pl.ds(0, C)], ssem[b]
            ).wait()

        for b in range(NBUF):
            issue_gather(b, b)

        ngrp = nch // NBUF

        def body(g, carry):
            for b in range(NBUF):
                q = g * NBUF + b
                wait_gather(b)

                @pl.when(g >= 1)
                def _():
                    wait_store(b)

                @plsc.parallel_loop(0, D, unroll=8)
                def _(d):
                    col_vec = jnp.full((_LANES,), 0, dtype=jnp.int32) + d
                    for jb in range(C // _LANES):
                        vals = plsc.load_gather(
                            rows_v.at[b], [row_vecs[jb], col_vec]
                        )
                        tr_v[b, d, pl.ds(jb * _LANES, _LANES)] = vals

                @pl.when(g < ngrp - 1)
                def _():
                    issue_gather(q + NBUF, b)

                issue_store(q, b)

            return carry

        lax.fori_loop(0, ngrp, body, 0)

        for b in range(NBUF):
            wait_store(b)

    out3 = gather_kernel(W_in, idxT)
    return jnp.transpose(out3, (2, 0, 1))


# speedup vs baseline: 1.0296x; 1.0296x over previous
import functools

import jax
import jax.numpy as jnp
from jax import lax
from jax.experimental import pallas as pl
from jax.experimental.pallas import tpu as pltpu
from jax.experimental.pallas import tpu_sc as plsc

_NC = 2
_NS = 16
_NW = _NC * _NS
_LANES = 16


def kernel(W_in, indices):
    V, D = W_in.shape
    B, L = indices.shape
    C = 128
    BPT = B // _NW
    NBB = BPT // C
    nch = L * NBB
    NBUF = 4
    assert BPT * _NW == B and NBB * C == BPT

    idxT = jnp.swapaxes(indices, 0, 1)

    mesh = plsc.VectorSubcoreMesh(core_axis_name="c", subcore_axis_name="s")

    @functools.partial(
        pl.kernel,
        mesh=mesh,
        out_type=jax.ShapeDtypeStruct((L, D, B), jnp.float32),
        scratch_types=[
            pltpu.VMEM((L, BPT), jnp.int32),
            pltpu.VMEM((NBUF, C, D), jnp.float32),
            pltpu.VMEM((NBUF, D, C), jnp.float32),
            [pltpu.SemaphoreType.DMA] * NBUF,
            [pltpu.SemaphoreType.DMA] * NBUF,
        ],
        compiler_params=pltpu.CompilerParams(
            use_tc_tiling_on_sc=False,
            needs_layout_passes=False,
            disable_bounds_checks=True,
        ),
    )
    def gather_kernel(table, idx, out, idx_v, rows_v, tr_v, gsem, ssem):
        wid = lax.axis_index("s") * _NC + lax.axis_index("c")
        col0 = wid * BPT
        pltpu.sync_copy(idx.at[:, pl.ds(col0, BPT)], idx_v)

        row_vecs = [
            lax.iota(jnp.int32, _LANES) + jb * _LANES
            for jb in range(C // _LANES)
        ]

        def issue_gather(q, b):
            l = q // NBB
            bb = q % NBB
            pltpu.async_copy(
                table.at[idx_v.at[l, pl.ds(bb * C, C)]], rows_v.at[b], gsem[b]
            )

        def wait_gather(b):
            pltpu.make_async_copy(
                table.at[idx_v.at[0, pl.ds(0, C)]], rows_v.at[b], gsem[b]
            ).wait()

        def issue_store(q, b):
            l = q // NBB
            bb = q % NBB
            pltpu.async_copy(
                tr_v.at[b], out.at[l, :, pl.ds(col0 + bb * C, C)], ssem[b]
            )

        def wait_store(b):
            pltpu.make_async_copy(
                tr_v.at[b], out.at[0, :, pl.ds(0, C)], ssem[b]
            ).wait()

        for b in range(NBUF):
            issue_gather(b, b)

        ngrp = nch // NBUF

        def body(g, carry):
            for b in range(NBUF):
                q = g * NBUF + b
                wait_gather(b)

                @pl.when(g >= 1)
                def _():
                    wait_store(b)

                @plsc.parallel_loop(0, D, unroll=4)
                def _(d):
                    col_vec = jnp.full((_LANES,), 0, dtype=jnp.int32) + d
                    for jb in range(C // _LANES):
                        vals = plsc.load_gather(
                            rows_v.at[b], [row_vecs[jb], col_vec]
                        )
                        tr_v[b, d, pl.ds(jb * _LANES, _LANES)] = vals

                @pl.when(g < ngrp - 1)
                def _():
                    issue_gather(q + NBUF, b)

                issue_store(q, b)

            return carry

        lax.fori_loop(0, ngrp, body, 0)

        for b in range(NBUF):
            wait_store(b)

    out3 = gather_kernel(W_in, idxT)
    return jnp.transpose(out3, (2, 0, 1))


# speedup vs baseline: 1.1871x; 1.1529x over previous
import functools

import jax
import jax.numpy as jnp
from jax import lax
from jax.experimental import pallas as pl
from jax.experimental.pallas import tpu as pltpu
from jax.experimental.pallas import tpu_sc as plsc

_NC = 2
_NS = 16
_NW = _NC * _NS
_LANES = 16


def kernel(W_in, indices):
    V, D = W_in.shape
    B, L = indices.shape
    C = 128
    BPT = B // _NW
    NBB = BPT // C
    nch = L * NBB
    NBUF = 2
    NJB = C // _LANES
    TD, DI = D // 8, 8
    assert BPT * _NW == B and NBB * C == BPT and D == 64

    idxT = jnp.swapaxes(indices, 0, 1)
    W2 = W_in.reshape(V // 2, 2 * D)

    mesh = plsc.VectorSubcoreMesh(core_axis_name="c", subcore_axis_name="s")

    @functools.partial(
        pl.kernel,
        mesh=mesh,
        out_type=jax.ShapeDtypeStruct((L, TD, B // C, DI, C), jnp.float32),
        scratch_types=[
            pltpu.VMEM((L, BPT), jnp.int32),
            pltpu.VMEM((NBUF, C), jnp.int32),
            pltpu.VMEM((NBUF, C, 2 * D), jnp.float32),
            pltpu.VMEM((NBUF, TD, DI, C), jnp.float32),
            [pltpu.SemaphoreType.DMA] * NBUF,
            [pltpu.SemaphoreType.DMA] * NBUF,
        ],
        compiler_params=pltpu.CompilerParams(
            use_tc_tiling_on_sc=False,
            needs_layout_passes=False,
            disable_bounds_checks=True,
        ),
    )
    def gather_kernel(table, idx, out, idx_v, idx2_v, rows_v, tr_v, gsem, ssem):
        wid = lax.axis_index("s") * _NC + lax.axis_index("c")
        col0 = wid * BPT
        pltpu.sync_copy(idx.at[:, pl.ds(col0, BPT)], idx_v)

        row_vecs = [
            lax.iota(jnp.int32, _LANES) + jb * _LANES for jb in range(NJB)
        ]

        def stage_pair_indices(q, b):
            l = q // NBB
            bb = q % NBB
            for jb in range(NJB):
                v = idx_v[l, pl.ds(bb * C + jb * _LANES, _LANES)]
                idx2_v[b, pl.ds(jb * _LANES, _LANES)] = v >> 1

        def issue_gather(b):
            pltpu.async_copy(
                table.at[idx2_v.at[b]], rows_v.at[b], gsem[b]
            )

        def wait_gather(b):
            pltpu.make_async_copy(
                table.at[idx2_v.at[b]], rows_v.at[b], gsem[b]
            ).wait()

        def issue_store(q, b):
            l = q // NBB
            tbg = wid * NBB + q % NBB
            pltpu.async_copy(tr_v.at[b], out.at[l, :, tbg], ssem[b])

        def wait_store(b):
            pltpu.make_async_copy(tr_v.at[b], out.at[0, :, 0], ssem[b]).wait()

        for b in range(NBUF):
            stage_pair_indices(b, b)
            issue_gather(b)

        ngrp = nch // NBUF

        def body(g, carry):
            for b in range(NBUF):
                q = g * NBUF + b
                l = q // NBB
                bb = q % NBB
                wait_gather(b)

                @pl.when(g >= 1)
                def _():
                    wait_store(b)

                halves = [
                    (idx_v[l, pl.ds(bb * C + jb * _LANES, _LANES)] & 1) << 6
                    for jb in range(NJB)
                ]

                @plsc.parallel_loop(0, D, unroll=4)
                def _(d):
                    td = d // DI
                    di = d % DI
                    for jb in range(NJB):
                        vals = plsc.load_gather(
                            rows_v.at[b], [row_vecs[jb], halves[jb] + d]
                        )
                        tr_v[b, td, di, pl.ds(jb * _LANES, _LANES)] = vals

                @pl.when(g < ngrp - 1)
                def _():
                    stage_pair_indices(q + NBUF, b)
                    issue_gather(b)

                issue_store(q, b)

            return carry

        lax.fori_loop(0, ngrp, body, 0)

        for b in range(NBUF):
            wait_store(b)

    out5 = gather_kernel(W2, idxT)
    return jnp.transpose(out5, (2, 4, 0, 1, 3)).reshape(B, L, D)


# speedup vs baseline: 1.5879x; 1.3377x over previous
import functools

import jax
import jax.numpy as jnp
from jax import lax
from jax.experimental import pallas as pl
from jax.experimental.pallas import tpu as pltpu
from jax.experimental.pallas import tpu_sc as plsc

_NC = 2
_NS = 16
_NW = _NC * _NS
_LANES = 16


def kernel(W_in, indices):
    V, D = W_in.shape
    B, L = indices.shape
    C = 128
    BPT = B // _NW
    NBB = BPT // C
    nch = L * NBB
    NBUF = 2
    NJB = C // _LANES
    TD, DI = D // 8, 8
    assert BPT * _NW == B and NBB * C == BPT and D == 64

    idxT = jnp.swapaxes(indices, 0, 1)
    W2 = W_in.reshape(V // 2, 2 * D)

    mesh = plsc.VectorSubcoreMesh(core_axis_name="c", subcore_axis_name="s")

    @functools.partial(
        pl.kernel,
        mesh=mesh,
        out_type=jax.ShapeDtypeStruct((L, TD, B // C, DI, C), jnp.float32),
        scratch_types=[
            pltpu.VMEM((L, BPT), jnp.int32),
            pltpu.VMEM((NBUF, C), jnp.int32),
            pltpu.VMEM((NBUF, C, 2 * D), jnp.float32),
            pltpu.VMEM((NBUF, TD, DI, C), jnp.float32),
            [pltpu.SemaphoreType.DMA] * NBUF,
            [pltpu.SemaphoreType.DMA] * NBUF,
        ],
        compiler_params=pltpu.CompilerParams(
            use_tc_tiling_on_sc=False,
            needs_layout_passes=False,
            disable_bounds_checks=True,
        ),
    )
    def gather_kernel(table, idx, out, idx_v, idx2_v, rows_v, tr_v, gsem, ssem):
        wid = lax.axis_index("s") * _NC + lax.axis_index("c")
        col0 = wid * BPT
        pltpu.sync_copy(idx.at[:, pl.ds(col0, BPT)], idx_v)

        row_vecs = [
            lax.iota(jnp.int32, _LANES) + jb * _LANES for jb in range(NJB)
        ]

        def stage_pair_indices(q, b):
            l = q // NBB
            bb = q % NBB
            for jb in range(NJB):
                v = idx_v[l, pl.ds(bb * C + jb * _LANES, _LANES)]
                idx2_v[b, pl.ds(jb * _LANES, _LANES)] = v >> 1

        def issue_gather(b):
            pltpu.async_copy(
                table.at[idx2_v.at[b]], rows_v.at[b], gsem[b]
            )

        def wait_gather(b):
            pltpu.make_async_copy(
                table.at[idx2_v.at[b]], rows_v.at[b], gsem[b]
            ).wait()

        def issue_store(q, b):
            l = q // NBB
            tbg = wid * NBB + q % NBB
            pltpu.async_copy(tr_v.at[b], out.at[l, :, tbg], ssem[b])

        def wait_store(b):
            pltpu.make_async_copy(tr_v.at[b], out.at[0, :, 0], ssem[b]).wait()

        for b in range(NBUF):
            stage_pair_indices(b, b)
            issue_gather(b)

        ngrp = nch // NBUF

        def body(g, carry):
            for b in range(NBUF):
                q = g * NBUF + b
                l = q // NBB
                bb = q % NBB
                wait_gather(b)

                @pl.when(g >= 1)
                def _():
                    wait_store(b)

                halves = [
                    (idx_v[l, pl.ds(bb * C + jb * _LANES, _LANES)] & 1) << 6
                    for jb in range(NJB)
                ]

                iota = lax.iota(jnp.int32, _LANES)
                for kb in range(D // _LANES):
                    colbases = [h + kb * _LANES for h in halves]

                    @plsc.parallel_loop(0, _LANES, unroll=4)
                    def _(s, kb=kb, colbases=colbases):
                        rot = (iota + s) & (_LANES - 1)
                        td_vec = (rot >> 3) + 2 * kb
                        di_vec = rot & 7
                        for jb in range(NJB):
                            vals = plsc.load_gather(
                                rows_v.at[b], [row_vecs[jb], colbases[jb] + rot]
                            )
                            plsc.store_scatter(
                                tr_v.at[b],
                                [td_vec, di_vec, row_vecs[jb]],
                                vals,
                            )

                @pl.when(g < ngrp - 1)
                def _():
                    stage_pair_indices(q + NBUF, b)
                    issue_gather(b)

                issue_store(q, b)

            return carry

        lax.fori_loop(0, ngrp, body, 0)

        for b in range(NBUF):
            wait_store(b)

    out5 = gather_kernel(W2, idxT)
    return jnp.transpose(out5, (2, 4, 0, 1, 3)).reshape(B, L, D)


# speedup vs baseline: 1.6095x; 1.0136x over previous
import functools

import jax
import jax.numpy as jnp
from jax import lax
from jax.experimental import pallas as pl
from jax.experimental.pallas import tpu as pltpu
from jax.experimental.pallas import tpu_sc as plsc

_NC = 2
_NS = 16
_NW = _NC * _NS
_LANES = 16


def kernel(W_in, indices):
    V, D = W_in.shape
    B, L = indices.shape
    C = 128
    BPT = B // _NW
    NBB = BPT // C
    nch = L * NBB
    NBUF = 4
    NJB = C // _LANES
    TD, DI = D // 8, 8
    assert BPT * _NW == B and NBB * C == BPT and D == 64

    idxT = jnp.swapaxes(indices, 0, 1)
    W2 = W_in.reshape(V // 2, 2 * D)

    mesh = plsc.VectorSubcoreMesh(core_axis_name="c", subcore_axis_name="s")

    @functools.partial(
        pl.kernel,
        mesh=mesh,
        out_type=jax.ShapeDtypeStruct((L, TD, B // C, DI, C), jnp.float32),
        scratch_types=[
            pltpu.VMEM((L, BPT), jnp.int32),
            pltpu.VMEM((NBUF, C), jnp.int32),
            pltpu.VMEM((NBUF, C, 2 * D), jnp.float32),
            pltpu.VMEM((NBUF, TD, DI, C), jnp.float32),
            [pltpu.SemaphoreType.DMA] * NBUF,
            [pltpu.SemaphoreType.DMA] * NBUF,
        ],
        compiler_params=pltpu.CompilerParams(
            use_tc_tiling_on_sc=False,
            needs_layout_passes=False,
            disable_bounds_checks=True,
        ),
    )
    def gather_kernel(table, idx, out, idx_v, idx2_v, rows_v, tr_v, gsem, ssem):
        wid = lax.axis_index("s") * _NC + lax.axis_index("c")
        col0 = wid * BPT
        pltpu.sync_copy(idx.at[:, pl.ds(col0, BPT)], idx_v)

        row_vecs = [
            lax.iota(jnp.int32, _LANES) + jb * _LANES for jb in range(NJB)
        ]

        def stage_pair_indices(q, b):
            l = q // NBB
            bb = q % NBB
            for jb in range(NJB):
                v = idx_v[l, pl.ds(bb * C + jb * _LANES, _LANES)]
                idx2_v[b, pl.ds(jb * _LANES, _LANES)] = v >> 1

        def issue_gather(b):
            pltpu.async_copy(
                table.at[idx2_v.at[b]], rows_v.at[b], gsem[b]
            )

        def wait_gather(b):
            pltpu.make_async_copy(
                table.at[idx2_v.at[b]], rows_v.at[b], gsem[b]
            ).wait()

        def issue_store(q, b):
            l = q // NBB
            tbg = wid * NBB + q % NBB
            pltpu.async_copy(tr_v.at[b], out.at[l, :, tbg], ssem[b])

        def wait_store(b):
            pltpu.make_async_copy(tr_v.at[b], out.at[0, :, 0], ssem[b]).wait()

        for b in range(NBUF):
            stage_pair_indices(b, b)
            issue_gather(b)

        ngrp = nch // NBUF

        def body(g, carry):
            for b in range(NBUF):
                q = g * NBUF + b
                l = q // NBB
                bb = q % NBB
                wait_gather(b)

                @pl.when(g >= 1)
                def _():
                    wait_store(b)

                halves = [
                    (idx_v[l, pl.ds(bb * C + jb * _LANES, _LANES)] & 1) << 6
                    for jb in range(NJB)
                ]

                iota = lax.iota(jnp.int32, _LANES)
                for kb in range(D // _LANES):
                    colbases = [h + kb * _LANES for h in halves]

                    @plsc.parallel_loop(0, _LANES, unroll=4)
                    def _(s, kb=kb, colbases=colbases):
                        rot = (iota + s) & (_LANES - 1)
                        td_vec = (rot >> 3) + 2 * kb
                        di_vec = rot & 7
                        for jb in range(NJB):
                            vals = plsc.load_gather(
                                rows_v.at[b], [row_vecs[jb], colbases[jb] + rot]
                            )
                            plsc.store_scatter(
                                tr_v.at[b],
                                [td_vec, di_vec, row_vecs[jb]],
                                vals,
                            )

                @pl.when(g < ngrp - 1)
                def _():
                    stage_pair_indices(q + NBUF, b)
                    issue_gather(b)

                issue_store(q, b)

            return carry

        lax.fori_loop(0, ngrp, body, 0)

        for b in range(NBUF):
            wait_store(b)

    out5 = gather_kernel(W2, idxT)
    return jnp.transpose(out5, (2, 4, 0, 1, 3)).reshape(B, L, D)


# speedup vs baseline: 1.8689x; 1.1612x over previous
import functools

import jax
import jax.numpy as jnp
from jax import lax
from jax.experimental import pallas as pl
from jax.experimental.pallas import tpu as pltpu
from jax.experimental.pallas import tpu_sc as plsc

_NC = 2
_NS = 16
_NW = _NC * _NS
_LANES = 16


def kernel(W_in, indices):
    V, D = W_in.shape
    B, L = indices.shape
    C = 128
    BPT = B // _NW
    NBB = BPT // C
    nch = L * NBB
    NBUF = 4
    NJB = C // _LANES
    TD, DI = D // 8, 8
    assert BPT * _NW == B and NBB * C == BPT and D == 64

    idxT = jnp.swapaxes(indices, 0, 1)

    mesh = plsc.VectorSubcoreMesh(core_axis_name="c", subcore_axis_name="s")

    @functools.partial(
        pl.kernel,
        mesh=mesh,
        out_type=jax.ShapeDtypeStruct((L, TD, B // C, DI, C), jnp.float32),
        scratch_types=[
            pltpu.VMEM((L, BPT), jnp.int32),
            pltpu.VMEM((NBUF, C, D), jnp.float32),
            pltpu.VMEM((NBUF, TD, DI, C), jnp.float32),
            [pltpu.SemaphoreType.DMA] * NBUF,
            [pltpu.SemaphoreType.DMA] * NBUF,
        ],
        compiler_params=pltpu.CompilerParams(
            use_tc_tiling_on_sc=False,
            needs_layout_passes=False,
            disable_bounds_checks=True,
        ),
    )
    def gather_kernel(table, idx, out, idx_v, rows_v, tr_v, gsem, ssem):
        wid = lax.axis_index("s") * _NC + lax.axis_index("c")
        col0 = wid * BPT
        pltpu.sync_copy(idx.at[:, pl.ds(col0, BPT)], idx_v)

        row_vecs = [
            lax.iota(jnp.int32, _LANES) + jb * _LANES for jb in range(NJB)
        ]

        def issue_gather(q, b):
            l = q // NBB
            bb = q % NBB
            pltpu.async_copy(
                table.at[idx_v.at[l, pl.ds(bb * C, C)]], rows_v.at[b], gsem[b]
            )

        def wait_gather(b):
            pltpu.make_async_copy(
                table.at[idx_v.at[0, pl.ds(0, C)]], rows_v.at[b], gsem[b]
            ).wait()

        def issue_store(q, b):
            l = q // NBB
            tbg = wid * NBB + q % NBB
            pltpu.async_copy(tr_v.at[b], out.at[l, :, tbg], ssem[b])

        def wait_store(b):
            pltpu.make_async_copy(tr_v.at[b], out.at[0, :, 0], ssem[b]).wait()

        for b in range(NBUF):
            issue_gather(b, b)

        ngrp = nch // NBUF

        def body(g, carry):
            for b in range(NBUF):
                q = g * NBUF + b
                wait_gather(b)

                @pl.when(g >= 1)
                def _():
                    wait_store(b)

                iota = lax.iota(jnp.int32, _LANES)
                for kb in range(D // _LANES):

                    @plsc.parallel_loop(0, _LANES, unroll=4)
                    def _(s, kb=kb):
                        rot = (iota + s) & (_LANES - 1)
                        td_vec = (rot >> 3) + 2 * kb
                        di_vec = rot & 7
                        cols = rot + kb * _LANES
                        for jb in range(NJB):
                            vals = plsc.load_gather(
                                rows_v.at[b], [row_vecs[jb], cols]
                            )
                            plsc.store_scatter(
                                tr_v.at[b],
                                [td_vec, di_vec, row_vecs[jb]],
                                vals,
                            )

                @pl.when(g < ngrp - 1)
                def _():
                    issue_gather(q + NBUF, b)

                issue_store(q, b)

            return carry

        lax.fori_loop(0, ngrp, body, 0)

        for b in range(NBUF):
            wait_store(b)

    out5 = gather_kernel(W_in, idxT)
    return jnp.transpose(out5, (2, 4, 0, 1, 3)).reshape(B, L, D)
